# baseline (speedup 1.0000x reference)
import functools

import jax
import jax.numpy as jnp
from jax import lax
from jax.experimental import pallas as pl
from jax.experimental.pallas import tpu as pltpu
from jax.experimental.pallas import tpu_sc as plsc

_NC = 2
_NS = 16
_NW = _NC * _NS
_CHUNK = 128
_L = 16


@functools.lru_cache(maxsize=None)
def _make_gather(V, D, B0, B1):
    B = B0 * B1
    n_blocks = B // (_NW * _CHUNK)
    n_tc = B0 // _CHUNK
    dsub = D // 8
    assert n_blocks % 2 == 0
    mesh = plsc.VectorSubcoreMesh(core_axis_name="c", subcore_axis_name="s")

    @functools.partial(
        pl.kernel,
        mesh=mesh,
        out_type=jax.ShapeDtypeStruct((B1, dsub, n_tc, 8, _CHUNK), jnp.float32),
        scratch_types=[
            pltpu.VMEM((n_blocks, _CHUNK), jnp.int32),
            pltpu.VMEM((2, _CHUNK, D), jnp.float32),
            pltpu.VMEM((2, dsub, 8, _CHUNK), jnp.float32),
            pltpu.SemaphoreType.DMA,
            pltpu.SemaphoreType.DMA,
            pltpu.SemaphoreType.DMA,
            pltpu.SemaphoreType.DMA,
        ],
        compiler_params=pltpu.CompilerParams(
            use_tc_tiling_on_sc=False, needs_layout_passes=False),
    )
    def k(idx_hbm, table_hbm, out_hbm, idx_v, rbuf, tbuf, g0, g1, w0, w1):
        gsems = (g0, g1)
        wsems = (w0, w1)
        wid = lax.axis_index("s") * _NC + lax.axis_index("c")
        pltpu.sync_copy(idx_hbm.at[wid], idx_v)
        iotav = lax.iota(jnp.int32, _L)

        def gather(j, b):
            return pltpu.async_copy(
                table_hbm.at[idx_v.at[j]], rbuf.at[b], gsems[b])

        def wait_gather(j, b):
            pltpu.make_async_copy(
                table_hbm.at[idx_v.at[j]], rbuf.at[b], gsems[b]).wait()

        def out_dst(j):
            p = wid * n_blocks + j
            b1 = lax.div(p, n_tc)
            tc = lax.rem(p, n_tc)
            return out_hbm.at[b1, pl.ds(0, dsub), tc]

        def write(j, b):
            return pltpu.async_copy(tbuf.at[b], out_dst(j), wsems[b])

        def wait_write(j, b):
            pltpu.make_async_copy(tbuf.at[b], out_dst(j), wsems[b]).wait()

        gather(0, 0)
        gather(1, 1)

        def body(g, carry):
            for b in range(2):
                j = 2 * g + b
                wait_gather(j, b)

                @pl.when(g >= 1)
                def _():
                    wait_write(j - 2, b)

                for gg in range(_CHUNK // _L):
                    rows = iotav + _L * gg
                    for f in range(D):
                        col = jnp.full((_L,), f, jnp.int32)
                        vec = plsc.load_gather(rbuf.at[b], [rows, col])
                        tbuf[b, f // 8, f % 8, pl.ds(_L * gg, _L)] = vec
                write(j, b)

                @pl.when(g < n_blocks // 2 - 1)
                def _():
                    gather(j + 2, b)

            return carry

        lax.fori_loop(0, n_blocks // 2, body, 0)
        wait_write(n_blocks - 2, 0)
        wait_write(n_blocks - 1, 1)

    return k


def kernel(x, table):
    B0, B1 = x.shape
    V, D = table.shape
    n_tc = B0 // _CHUNK
    n_blocks = B0 * B1 // (_NW * _CHUNK)
    idx = x.T.reshape(B1, n_tc, _CHUNK).reshape(_NW, n_blocks, _CHUNK)
    idx = idx.astype(jnp.int32)
    out5 = _make_gather(V, D, B0, B1)(idx, table)
    return out5.transpose(2, 4, 0, 1, 3).reshape(B0, B1, D)


# speedup vs baseline: 1.2601x; 1.2601x over previous
import functools

import jax
import jax.numpy as jnp
from jax import lax
from jax.experimental import pallas as pl
from jax.experimental.pallas import tpu as pltpu
from jax.experimental.pallas import tpu_sc as plsc

_NC = 2
_NS = 16
_NW = _NC * _NS
_CHUNK = 128
_L = 16


@functools.lru_cache(maxsize=None)
def _make_gather(V, D, B0, B1):
    B = B0 * B1
    n_blocks = B // (_NW * _CHUNK)
    n_tc = B0 // _CHUNK
    dsub = D // 8
    assert n_blocks % 2 == 0
    mesh = plsc.VectorSubcoreMesh(core_axis_name="c", subcore_axis_name="s")

    @functools.partial(
        pl.kernel,
        mesh=mesh,
        out_type=jax.ShapeDtypeStruct((B1, dsub, n_tc, 8, _CHUNK), jnp.float32),
        scratch_types=[
            pltpu.VMEM((n_blocks, _CHUNK), jnp.int32),
            pltpu.VMEM((2, _CHUNK, D), jnp.float32),
            pltpu.VMEM((2, dsub, 8, _CHUNK), jnp.float32),
            pltpu.SemaphoreType.DMA,
            pltpu.SemaphoreType.DMA,
            pltpu.SemaphoreType.DMA,
            pltpu.SemaphoreType.DMA,
        ],
        compiler_params=pltpu.CompilerParams(
            use_tc_tiling_on_sc=False, needs_layout_passes=False),
    )
    def k(idx_hbm, table_hbm, out_hbm, idx_v, rbuf, tbuf, g0, g1, w0, w1):
        gsems = (g0, g1)
        wsems = (w0, w1)
        wid = lax.axis_index("s") * _NC + lax.axis_index("c")
        pltpu.sync_copy(idx_hbm.at[wid], idx_v)
        iotav = lax.iota(jnp.int32, _L)

        def gather(j, b):
            return pltpu.async_copy(
                table_hbm.at[idx_v.at[j]], rbuf.at[b], gsems[b])

        def wait_gather(j, b):
            pltpu.make_async_copy(
                table_hbm.at[idx_v.at[j]], rbuf.at[b], gsems[b]).wait()

        def out_dst(j):
            p = wid * n_blocks + j
            b1 = lax.div(p, n_tc)
            tc = lax.rem(p, n_tc)
            return out_hbm.at[b1, pl.ds(0, dsub), tc]

        def write(j, b):
            return pltpu.async_copy(tbuf.at[b], out_dst(j), wsems[b])

        def wait_write(j, b):
            pltpu.make_async_copy(tbuf.at[b], out_dst(j), wsems[b]).wait()

        gather(0, 0)
        gather(1, 1)

        def body(g, carry):
            for b in range(2):
                j = 2 * g + b
                wait_gather(j, b)

                @pl.when(g >= 1)
                def _():
                    wait_write(j - 2, b)

                trv = []
                fiv = []
                for q in range(D // _L):
                    fv = iotav + _L * q
                    trv.append(lax.shift_right_logical(fv, 3))
                    fiv.append(lax.bitwise_and(fv, 7))
                for k in range(_CHUNK):
                    biv = jnp.full((_L,), k, jnp.int32)
                    for q in range(D // _L):
                        vec = rbuf[b, k, pl.ds(_L * q, _L)]
                        plsc.store_scatter(tbuf.at[b], [trv[q], fiv[q], biv], vec)
                write(j, b)

                @pl.when(g < n_blocks // 2 - 1)
                def _():
                    gather(j + 2, b)

            return carry

        lax.fori_loop(0, n_blocks // 2, body, 0)
        wait_write(n_blocks - 2, 0)
        wait_write(n_blocks - 1, 1)

    return k


def kernel(x, table):
    B0, B1 = x.shape
    V, D = table.shape
    n_tc = B0 // _CHUNK
    n_blocks = B0 * B1 // (_NW * _CHUNK)
    idx = x.T.reshape(B1, n_tc, _CHUNK).reshape(_NW, n_blocks, _CHUNK)
    idx = idx.astype(jnp.int32)
    out5 = _make_gather(V, D, B0, B1)(idx, table)
    return out5.transpose(2, 4, 0, 1, 3).reshape(B0, B1, D)


# speedup vs baseline: 1.6457x; 1.3060x over previous
import functools

import jax
import jax.numpy as jnp
from jax import lax
from jax.experimental import pallas as pl
from jax.experimental.pallas import tpu as pltpu
from jax.experimental.pallas import tpu_sc as plsc

_NC = 2
_NS = 16
_NW = _NC * _NS
_CHUNK = 128
_NBUF = 8


@functools.lru_cache(maxsize=None)
def _make_gather(V, D, B):
    b_per_w = B // _NW
    n_chunks = b_per_w // _CHUNK
    n_outer = n_chunks // _NBUF
    assert n_chunks % _NBUF == 0 and n_outer >= 2
    mesh = plsc.VectorSubcoreMesh(core_axis_name="c", subcore_axis_name="s")

    @functools.partial(
        pl.kernel,
        mesh=mesh,
        out_type=jax.ShapeDtypeStruct((B, D), jnp.float32),
        scratch_types=[
            pltpu.VMEM((n_chunks, _CHUNK), jnp.int32),
            pltpu.VMEM((_NBUF, _CHUNK, D), jnp.float32),
        ] + [pltpu.SemaphoreType.DMA] * _NBUF,
        compiler_params=pltpu.CompilerParams(use_tc_tiling_on_sc=False),
    )
    def k(idx_hbm, table_hbm, out_hbm, idx_v, rows_v, *sems):
        wid = lax.axis_index("s") * _NC + lax.axis_index("c")
        base = wid * b_per_w
        pltpu.sync_copy(idx_hbm.at[wid], idx_v)

        def gather(j, b):
            return pltpu.async_copy(table_hbm.at[idx_v.at[j]], rows_v.at[b], sems[b])

        for b in range(_NBUF):
            gather(b, b)

        def body(g, carry):
            for b in range(_NBUF):
                j = g * _NBUF + b
                pltpu.make_async_copy(table_hbm.at[idx_v.at[j]], rows_v.at[b], sems[b]).wait()
                pltpu.sync_copy(rows_v.at[b], out_hbm.at[pl.ds(base + j * _CHUNK, _CHUNK)])
                gather(j + _NBUF, b)
            return carry

        lax.fori_loop(0, n_outer - 1, body, 0)

        for b in range(_NBUF):
            j = (n_outer - 1) * _NBUF + b
            pltpu.make_async_copy(table_hbm.at[idx_v.at[j]], rows_v.at[b], sems[b]).wait()
            pltpu.sync_copy(rows_v.at[b], out_hbm.at[pl.ds(base + j * _CHUNK, _CHUNK)])

    return k


def kernel(x, table):
    B0, B1 = x.shape
    V, D = table.shape
    B = B0 * B1
    xf = x.reshape(_NW, (B // _NW) // _CHUNK, _CHUNK).astype(jnp.int32)
    out = _make_gather(V, D, B)(xf, table)
    return out.reshape(B0, B1, D)
